# Initial kernel scaffold; baseline (speedup 1.0000x reference)
import jax
import jax.numpy as jnp
from jax.experimental import pallas as pl


def kernel(node_feats, edge_feats, edge_index, node_graph_ids, samples, gc_pn_W, gc_pn_b, gc_pe1_W, gc_pe1_b, gc_pe2_W, gc_pe2_b, gc_et_W, gc_et_b, gc_gru_Wih, gc_gru_Whh, gc_gru_bih, gc_gru_bhh, gl_pe_W, gl_pe_b, gl_pn_W, gl_pn_b, gl_gru_Wih, gl_gru_Whh, gl_gru_bih, gl_gru_bhh, ro0_cl_W, ro0_cl_b, ro0_pn_W, ro0_pn_b, ro0_gru_Wih, ro0_gru_Whh, ro0_gru_bih, ro0_gru_bhh, ro1_cl_W, ro1_cl_b, ro1_pn_W, ro1_pn_b, ro1_gru_Wih, ro1_gru_Whh, ro1_gru_bih, ro1_gru_bhh, pred_W, pred_b):
    raise NotImplementedError("write your pallas kernel here")


# speedup vs baseline: 3.0975x; 3.0975x over previous
import functools

import jax
import jax.numpy as jnp
from jax import lax
from jax.experimental import pallas as pl
from jax.experimental.pallas import tpu as pltpu
from jax.experimental.pallas import tpu_sc as plsc

N = 10000
E = 160000
G = 512
S = 1024
NFS = 128
EFS = 16
GD = 200
D = 208
NV = D // 16
NW = 32
E_PAD = 163840
EPT = E_PAD // 16
C = 64
NCHUNK = EPT // C
NA = N + 16
LW = 112
LV = LW // 16
RPT = 632
RPT_LAST = NA - 15 * RPT

NBLK = 1000
EBLK = 2048
TINY = 1e-30


def _lrelu(x):
    return jnp.maximum(x, 0.01 * x)


def _elu(x):
    return jnp.where(x > 0, x, jnp.exp(jnp.minimum(x, 0.0)) - 1.0)


def _tc1_node_body(hv, pnWT, pnb, W1aT, Wa, pe2b, hv_new_o, hx_o, q1_o):
    hvb = hv[...]
    hv_new = _lrelu(jnp.dot(hvb, pnWT[...], preferred_element_type=jnp.float32)
                    + pnb[...])
    hv_new_o[...] = hv_new
    hx_o[...] = jnp.dot(hvb, W1aT[...], preferred_element_type=jnp.float32)
    q1_o[...] = jnp.dot(hv_new, Wa[...], preferred_element_type=jnp.float32) + pe2b[...]


def _tc1_node(hv, pnWT, pnb, W1aT, Wa, pe2b):
    nblk = N // NBLK
    return pl.pallas_call(
        _tc1_node_body,
        grid=(nblk,),
        in_specs=[
            pl.BlockSpec((NBLK, NFS), lambda i: (i, 0)),
            pl.BlockSpec((NFS, GD), lambda i: (0, 0)),
            pl.BlockSpec((1, GD), lambda i: (0, 0)),
            pl.BlockSpec((NFS, D), lambda i: (0, 0)),
            pl.BlockSpec((GD, 1), lambda i: (0, 0)),
            pl.BlockSpec((1, 1), lambda i: (0, 0)),
        ],
        out_specs=[
            pl.BlockSpec((NBLK, GD), lambda i: (i, 0)),
            pl.BlockSpec((NBLK, D), lambda i: (i, 0)),
            pl.BlockSpec((NBLK, 1), lambda i: (i, 0)),
        ],
        out_shape=[
            jax.ShapeDtypeStruct((N, GD), jnp.float32),
            jax.ShapeDtypeStruct((N, D), jnp.float32),
            jax.ShapeDtypeStruct((N, 1), jnp.float32),
        ],
    )(hv, pnWT, pnb, W1aT, Wa, pe2b)


def _tc1_edge_body(ef, W1bT, b1, ex_o):
    ex_o[...] = jnp.dot(ef[...], W1bT[...], preferred_element_type=jnp.float32) + b1[...]


def _tc1_edge(ef, W1bT, b1):
    eblk = E_PAD // EBLK
    return pl.pallas_call(
        _tc1_edge_body,
        grid=(eblk,),
        in_specs=[
            pl.BlockSpec((EBLK, EFS), lambda i: (i, 0)),
            pl.BlockSpec((EFS, D), lambda i: (0, 0)),
            pl.BlockSpec((1, D), lambda i: (0, 0)),
        ],
        out_specs=pl.BlockSpec((EBLK, D), lambda i: (i, 0)),
        out_shape=jax.ShapeDtypeStruct((E_PAD, D), jnp.float32),
    )(ef, W1bT, b1)


def _sc_exp(y_in):
    y = y_in * jnp.float32(1.4426950408889634)
    t = y + jnp.float32(12582912.0)
    i_f = t - jnp.float32(12582912.0)
    g = (y - i_f) * jnp.float32(0.6931471805599453)
    p = jnp.full_like(g, 1.0 / 720.0)
    for c in (1.0 / 120.0, 1.0 / 24.0, 1.0 / 6.0, 0.5, 1.0, 1.0):
        p = p * g + jnp.float32(c)
    s = lax.bitcast_convert_type((i_f.astype(jnp.int32) + 127) << 23,
                                 jnp.float32)
    return p * s


def _sc_zero_and_stage(zbuf, acc, sid):
    def zrow(r, _):
        for j in range(LV):
            zbuf[r, pl.ds(16 * j, 16)] = jnp.zeros((16,), jnp.float32)
        return 0
    lax.fori_loop(0, C, zrow, 0)
    base = sid * RPT
    for k in range(RPT_LAST // C):
        pltpu.sync_copy(zbuf.at[pl.ds(0, C)], acc.at[pl.ds(base + C * k, C)])

    @pl.when(sid < 15)
    def _():
        for k in range(RPT_LAST // C, RPT // C):
            pltpu.sync_copy(zbuf.at[pl.ds(0, C)],
                            acc.at[pl.ds(base + C * k, C)])
        if RPT % C:
            pltpu.sync_copy(zbuf.at[pl.ds(0, RPT % C)],
                            acc.at[pl.ds(base + (RPT // C) * C, RPT % C)])

    @pl.when(sid == 15)
    def _():
        if RPT_LAST % C:
            pltpu.sync_copy(zbuf.at[pl.ds(0, RPT_LAST % C)],
                            acc.at[pl.ds(base + (RPT_LAST // C) * C, RPT_LAST % C)])


def _sc_copy_out(acc, out, core, sid):
    base = sid * RPT

    @pl.when(sid < 15)
    def _():
        pltpu.sync_copy(acc.at[pl.ds(base, RPT)],
                        out.at[core, pl.ds(base, RPT)])

    @pl.when(sid == 15)
    def _():
        pltpu.sync_copy(acc.at[pl.ds(base, RPT_LAST)],
                        out.at[core, pl.ds(base, RPT_LAST)])


def _sc1_body(src_hbm, dst_hbm, hx_hbm, ex_hbm, q1_hbm, wb_hbm,
              s1_out,
              acc, rbuf, ebuf, hbuf, q1t, src_i, dst_i, wbt, sem):
    core = lax.axis_index("c")
    sid = lax.axis_index("s")
    lane = lax.iota(jnp.int32, 16)
    mask8 = lane == 8

    _sc_zero_and_stage(hbuf, acc, sid)
    pltpu.sync_copy(q1_hbm, q1t)
    pltpu.sync_copy(wb_hbm, wbt)
    plsc.subcore_barrier()

    def chunk(k, _):
        base = sid * EPT + k * C
        pltpu.sync_copy(src_hbm.at[pl.ds(base, C)], src_i)
        pltpu.sync_copy(dst_hbm.at[pl.ds(base, C)], dst_i)
        pltpu.async_copy(hx_hbm.at[src_i], rbuf, sem).wait()
        pltpu.sync_copy(ex_hbm.at[pl.ds(base, C)], ebuf)

        def group(t, _):
            dstv = dst_i[pl.ds(16 * t, 16)]
            qv = plsc.load_gather(q1t, [dstv])
            for j in range(16):
                r = 16 * t + j
                accv = jnp.zeros((16,), jnp.float32)
                vs = []
                for jj in range(NV):
                    v = rbuf[r, pl.ds(16 * jj, 16)] + ebuf[r, pl.ds(16 * jj, 16)]
                    v = jnp.maximum(v, 0.01 * v)
                    vs.append(v)
                    accv = accv + v * wbt[pl.ds(16 * jj, 16)]
                z = qv[j] + jnp.sum(accv)
                z = jnp.maximum(z, 0.01 * z)
                ezv = _sc_exp(jnp.full((16,), z, jnp.float32))

                @pl.when(core == 0)
                def _():
                    for jj in range(LV):
                        hbuf[r, pl.ds(16 * jj, 16)] = vs[jj] * ezv

                @pl.when(core == 1)
                def _():
                    for jj in range(LV - 1):
                        hbuf[r, pl.ds(16 * jj, 16)] = vs[6 + jj] * ezv
                    hbuf[r, pl.ds(16 * (LV - 1), 16)] = jnp.where(
                        mask8, ezv, vs[6 + LV - 1] * ezv)
            return 0

        lax.fori_loop(0, C // 16, group, 0)
        pltpu.sync_copy(hbuf, acc.at[dst_i], add=True)
        return 0

    lax.fori_loop(0, NCHUNK, chunk, 0)
    plsc.subcore_barrier()
    _sc_copy_out(acc, s1_out, core, sid)


@functools.cache
def _sc1_call():
    return pl.kernel(
        _sc1_body,
        out_type=jax.ShapeDtypeStruct((2, NA, LW), jnp.float32),
        mesh=plsc.VectorSubcoreMesh(core_axis_name="c", subcore_axis_name="s"),
        compiler_params=pltpu.CompilerParams(needs_layout_passes=False, use_tc_tiling_on_sc=False),
        scratch_types=[
            pltpu.VMEM_SHARED((NA, LW), jnp.float32),
            pltpu.VMEM((C, D), jnp.float32),
            pltpu.VMEM((C, D), jnp.float32),
            pltpu.VMEM((C, LW), jnp.float32),
            pltpu.VMEM((NA,), jnp.float32),
            pltpu.VMEM((C,), jnp.int32),
            pltpu.VMEM((C,), jnp.int32),
            pltpu.VMEM((D,), jnp.float32),
            pltpu.SemaphoreType.DMA,
        ],
    )


def _sc1(src, dst, hx, EX, q1, wb):
    return _sc1_call()(src, dst, hx, EX, q1, wb)


def _tc2_body(s1p, hv_new, etWT, etb,
              WirT, WizT, WinT, WhrT, WhzT, WhnT, bi3, bh3,
              Wd, Ws, peb, pnT0, pnb0, pnT1, pnb1,
              nf_o, qd_o, qs_o, hvp0_o, hvp1_o):
    s1 = s1p[...]
    e200 = (lax.broadcasted_iota(jnp.int32, (D, 1), 0) == GD).astype(jnp.float32)
    s = jnp.dot(s1, e200, preferred_element_type=jnp.float32)
    good = s > 0
    S1 = jnp.where(good, s1 / jnp.maximum(s, TINY), 0.0)
    ind = good.astype(jnp.float32)
    ctx = _elu(jnp.dot(S1, etWT[...], preferred_element_type=jnp.float32)
               + ind * etb[...])
    h = hv_new[...]
    bi = bi3[...]
    bh = bh3[...]
    ir = jnp.dot(ctx, WirT[...], preferred_element_type=jnp.float32) + bi[:, 0:GD]
    iz = jnp.dot(ctx, WizT[...], preferred_element_type=jnp.float32) + bi[:, GD:2 * GD]
    inn = jnp.dot(ctx, WinT[...], preferred_element_type=jnp.float32) + bi[:, 2 * GD:]
    hr = jnp.dot(h, WhrT[...], preferred_element_type=jnp.float32) + bh[:, 0:GD]
    hz = jnp.dot(h, WhzT[...], preferred_element_type=jnp.float32) + bh[:, GD:2 * GD]
    hn = jnp.dot(h, WhnT[...], preferred_element_type=jnp.float32) + bh[:, 2 * GD:]
    r = jax.nn.sigmoid(ir + hr)
    zz = jax.nn.sigmoid(iz + hz)
    nn = jnp.tanh(inn + r * hn)
    nf = jnp.maximum((1.0 - zz) * nn + zz * h, 0.0)
    nf_o[...] = nf
    qd_o[...] = jnp.dot(nf, Wd[...], preferred_element_type=jnp.float32) + peb[...]
    qs_o[...] = jnp.dot(nf, Ws[...], preferred_element_type=jnp.float32)
    hvp0_o[...] = jnp.dot(nf, pnT0[...], preferred_element_type=jnp.float32) + pnb0[...]
    hvp1_o[...] = jnp.dot(nf, pnT1[...], preferred_element_type=jnp.float32) + pnb1[...]


def _tc2(s1p, hv_new, etWT, etb, gru_w, Wd, Ws, peb, pnT0, pnb0, pnT1, pnb1):
    nblk = N // NBLK
    wspec = [
        pl.BlockSpec((GD, GD), lambda i: (0, 0)) for _ in range(6)
    ]
    return pl.pallas_call(
        _tc2_body,
        grid=(nblk,),
        in_specs=[
            pl.BlockSpec((NBLK, D), lambda i: (i, 0)),
            pl.BlockSpec((NBLK, GD), lambda i: (i, 0)),
            pl.BlockSpec((D, GD), lambda i: (0, 0)),
            pl.BlockSpec((1, GD), lambda i: (0, 0)),
            *wspec,
            pl.BlockSpec((1, 3 * GD), lambda i: (0, 0)),
            pl.BlockSpec((1, 3 * GD), lambda i: (0, 0)),
            pl.BlockSpec((GD, 1), lambda i: (0, 0)),
            pl.BlockSpec((GD, 1), lambda i: (0, 0)),
            pl.BlockSpec((1, 1), lambda i: (0, 0)),
            pl.BlockSpec((GD, LW), lambda i: (0, 0)),
            pl.BlockSpec((1, LW), lambda i: (0, 0)),
            pl.BlockSpec((GD, LW), lambda i: (0, 0)),
            pl.BlockSpec((1, LW), lambda i: (0, 0)),
        ],
        out_specs=[
            pl.BlockSpec((NBLK, GD), lambda i: (i, 0)),
            pl.BlockSpec((NBLK, 1), lambda i: (i, 0)),
            pl.BlockSpec((NBLK, 1), lambda i: (i, 0)),
            pl.BlockSpec((NBLK, LW), lambda i: (i, 0)),
            pl.BlockSpec((NBLK, LW), lambda i: (i, 0)),
        ],
        out_shape=[
            jax.ShapeDtypeStruct((N, GD), jnp.float32),
            jax.ShapeDtypeStruct((N, 1), jnp.float32),
            jax.ShapeDtypeStruct((N, 1), jnp.float32),
            jax.ShapeDtypeStruct((N, LW), jnp.float32),
            jax.ShapeDtypeStruct((N, LW), jnp.float32),
        ],
    )(s1p, hv_new, etWT, etb, *gru_w, Wd, Ws, peb, pnT0, pnb0, pnT1, pnb1)


def _sc2_body(src_hbm, dst_hbm, hvp0_hbm, hvp1_hbm, qd_hbm, qs_hbm,
              s2_out,
              acc, rbuf, qdt, qst, src_i, dst_i, sem):
    core = lax.axis_index("c")
    sid = lax.axis_index("s")
    lane = lax.iota(jnp.int32, 16)
    mask8 = lane == 8

    _sc_zero_and_stage(rbuf, acc, sid)
    pltpu.sync_copy(qd_hbm, qdt)
    pltpu.sync_copy(qs_hbm, qst)
    plsc.subcore_barrier()

    def chunk(k, _):
        base = sid * EPT + k * C
        pltpu.sync_copy(src_hbm.at[pl.ds(base, C)], src_i)
        pltpu.sync_copy(dst_hbm.at[pl.ds(base, C)], dst_i)

        @pl.when(core == 0)
        def _():
            pltpu.async_copy(hvp0_hbm.at[src_i], rbuf, sem).wait()

        @pl.when(core == 1)
        def _():
            pltpu.async_copy(hvp1_hbm.at[src_i], rbuf, sem).wait()

        def group(t, _):
            dstv = dst_i[pl.ds(16 * t, 16)]
            srcv = src_i[pl.ds(16 * t, 16)]
            zv = plsc.load_gather(qdt, [dstv]) + plsc.load_gather(qst, [srcv])
            zv = jnp.maximum(zv, 0.01 * zv)
            ez16 = _sc_exp(zv)
            for j in range(16):
                r = 16 * t + j
                ezv = jnp.full((16,), ez16[j], jnp.float32)

                @pl.when(core == 0)
                def _():
                    for jj in range(LV):
                        rbuf[r, pl.ds(16 * jj, 16)] = rbuf[r, pl.ds(16 * jj, 16)] * ezv

                @pl.when(core == 1)
                def _():
                    for jj in range(LV - 1):
                        rbuf[r, pl.ds(16 * jj, 16)] = rbuf[r, pl.ds(16 * jj, 16)] * ezv
                    rbuf[r, pl.ds(16 * (LV - 1), 16)] = jnp.where(
                        mask8, ezv, rbuf[r, pl.ds(16 * (LV - 1), 16)] * ezv)
            return 0

        lax.fori_loop(0, C // 16, group, 0)
        pltpu.sync_copy(rbuf, acc.at[dst_i], add=True)
        return 0

    lax.fori_loop(0, NCHUNK, chunk, 0)
    plsc.subcore_barrier()
    _sc_copy_out(acc, s2_out, core, sid)


@functools.cache
def _sc2_call():
    return pl.kernel(
        _sc2_body,
        out_type=jax.ShapeDtypeStruct((2, NA, LW), jnp.float32),
        mesh=plsc.VectorSubcoreMesh(core_axis_name="c", subcore_axis_name="s"),
        compiler_params=pltpu.CompilerParams(needs_layout_passes=False, use_tc_tiling_on_sc=False),
        scratch_types=[
            pltpu.VMEM_SHARED((NA, LW), jnp.float32),
            pltpu.VMEM((C, LW), jnp.float32),
            pltpu.VMEM((NA,), jnp.float32),
            pltpu.VMEM((NA,), jnp.float32),
            pltpu.VMEM((C,), jnp.int32),
            pltpu.VMEM((C,), jnp.int32),
            pltpu.SemaphoreType.DMA,
        ],
    )


def _sc2(src, dst, hvp0, hvp1, qd, qs):
    return _sc2_call()(src, dst, hvp0, hvp1, qd, qs)


def _tc3_body(s2p, nf, gid,
              WirT, WizT, WinT, WhrT, WhzT, WhnT, bi3, bh3,
              Wn0, clb0, pn0T, pn0b, Wn1, clb1, pn1T, pn1b,
              u0_o, hvr0_o, u1_o, hvr1_o, gf0_o):
    s2 = s2p[...]
    e200 = (lax.broadcasted_iota(jnp.int32, (D, 1), 0) == GD).astype(jnp.float32)
    s = jnp.dot(s2, e200, preferred_element_type=jnp.float32)
    ctx = _elu(jnp.where(s > 0, s2 / jnp.maximum(s, TINY), 0.0))
    h = nf[...]
    bi = bi3[...]
    bh = bh3[...]
    ir = jnp.dot(ctx, WirT[...], preferred_element_type=jnp.float32) + bi[:, 0:GD]
    iz = jnp.dot(ctx, WizT[...], preferred_element_type=jnp.float32) + bi[:, GD:2 * GD]
    inn = jnp.dot(ctx, WinT[...], preferred_element_type=jnp.float32) + bi[:, 2 * GD:]
    hr = jnp.dot(h, WhrT[...], preferred_element_type=jnp.float32) + bh[:, 0:GD]
    hz = jnp.dot(h, WhzT[...], preferred_element_type=jnp.float32) + bh[:, GD:2 * GD]
    hn = jnp.dot(h, WhnT[...], preferred_element_type=jnp.float32) + bh[:, 2 * GD:]
    r = jax.nn.sigmoid(ir + hr)
    zz = jax.nn.sigmoid(iz + hz)
    nn = jnp.tanh(inn + r * hn)
    nf2 = jnp.maximum((1.0 - zz) * nn + zz * h, 0.0)
    u0_o[...] = jnp.dot(nf2, Wn0[...], preferred_element_type=jnp.float32) + clb0[...]
    hvr0_o[...] = jnp.dot(nf2, pn0T[...], preferred_element_type=jnp.float32) + pn0b[...]
    u1_o[...] = jnp.dot(nf2, Wn1[...], preferred_element_type=jnp.float32) + clb1[...]
    hvr1_o[...] = jnp.dot(nf2, pn1T[...], preferred_element_type=jnp.float32) + pn1b[...]
    oh = (lax.broadcasted_iota(jnp.int32, (G, NBLK), 0)
          == gid[0, 0, :][None, :]).astype(jnp.float32)
    part = jnp.dot(oh, nf2, preferred_element_type=jnp.float32)

    @pl.when(pl.program_id(0) == 0)
    def _():
        gf0_o[...] = jnp.zeros_like(gf0_o)
    gf0_o[...] += part


def _tc3(s2p, nf, gid3, gru_w, ro_w):
    nblk = N // NBLK
    wspec = [pl.BlockSpec((GD if k < 3 else GD, GD), lambda i: (0, 0))
             for k in range(6)]
    wspec[0] = pl.BlockSpec((D, GD), lambda i: (0, 0))
    wspec[1] = pl.BlockSpec((D, GD), lambda i: (0, 0))
    wspec[2] = pl.BlockSpec((D, GD), lambda i: (0, 0))
    return pl.pallas_call(
        _tc3_body,
        grid=(nblk,),
        in_specs=[
            pl.BlockSpec((NBLK, D), lambda i: (i, 0)),
            pl.BlockSpec((NBLK, GD), lambda i: (i, 0)),
            pl.BlockSpec((1, 1, NBLK), lambda i: (i, 0, 0)),
            *wspec,
            pl.BlockSpec((1, 3 * GD), lambda i: (0, 0)),
            pl.BlockSpec((1, 3 * GD), lambda i: (0, 0)),
            pl.BlockSpec((GD, 1), lambda i: (0, 0)),
            pl.BlockSpec((1, 1), lambda i: (0, 0)),
            pl.BlockSpec((GD, GD), lambda i: (0, 0)),
            pl.BlockSpec((1, GD), lambda i: (0, 0)),
            pl.BlockSpec((GD, 1), lambda i: (0, 0)),
            pl.BlockSpec((1, 1), lambda i: (0, 0)),
            pl.BlockSpec((GD, GD), lambda i: (0, 0)),
            pl.BlockSpec((1, GD), lambda i: (0, 0)),
        ],
        out_specs=[
            pl.BlockSpec((NBLK, 1), lambda i: (i, 0)),
            pl.BlockSpec((NBLK, GD), lambda i: (i, 0)),
            pl.BlockSpec((NBLK, 1), lambda i: (i, 0)),
            pl.BlockSpec((NBLK, GD), lambda i: (i, 0)),
            pl.BlockSpec((G, GD), lambda i: (0, 0)),
        ],
        out_shape=[
            jax.ShapeDtypeStruct((N, 1), jnp.float32),
            jax.ShapeDtypeStruct((N, GD), jnp.float32),
            jax.ShapeDtypeStruct((N, 1), jnp.float32),
            jax.ShapeDtypeStruct((N, GD), jnp.float32),
            jax.ShapeDtypeStruct((G, GD), jnp.float32),
        ],
    )(s2p, nf, gid3, *gru_w, *ro_w)


def _tc4_body(gf, Wg, u, hvr, gid,
              WirT, WizT, WinT, WhrT, WhzT, WhnT, bi3, bh3,
              gfn_o, grp_acc, sz_acc):
    i = pl.program_id(0)

    @pl.when(i == 0)
    def _():
        grp_acc[...] = jnp.zeros_like(grp_acc)
        sz_acc[...] = jnp.zeros_like(sz_acc)

    gfb = gf[...]
    zg = jnp.dot(jnp.maximum(gfb, 0.0), Wg[...],
                 preferred_element_type=jnp.float32)
    ohT = (gid[0, 0, :][:, None]
           == lax.broadcasted_iota(jnp.int32, (NBLK, G), 1)).astype(jnp.float32)
    zn = jnp.dot(ohT, zg, preferred_element_type=jnp.float32) + u[...]
    ez = jnp.exp(_lrelu(zn))
    sz_acc[...] += jnp.dot(ohT.T, ez, preferred_element_type=jnp.float32)
    grp_acc[...] += jnp.dot(ohT.T, hvr[...] * ez,
                            preferred_element_type=jnp.float32)

    @pl.when(i == pl.num_programs(0) - 1)
    def _():
        sz = sz_acc[...]
        gr = jnp.where(sz > 0, grp_acc[...] / jnp.maximum(sz, TINY), 0.0)
        x = _elu(gr)
        bi = bi3[...]
        bh = bh3[...]
        ir = jnp.dot(x, WirT[...], preferred_element_type=jnp.float32) + bi[:, 0:GD]
        iz = jnp.dot(x, WizT[...], preferred_element_type=jnp.float32) + bi[:, GD:2 * GD]
        inn = jnp.dot(x, WinT[...], preferred_element_type=jnp.float32) + bi[:, 2 * GD:]
        hr = jnp.dot(gfb, WhrT[...], preferred_element_type=jnp.float32) + bh[:, 0:GD]
        hz = jnp.dot(gfb, WhzT[...], preferred_element_type=jnp.float32) + bh[:, GD:2 * GD]
        hn = jnp.dot(gfb, WhnT[...], preferred_element_type=jnp.float32) + bh[:, 2 * GD:]
        r = jax.nn.sigmoid(ir + hr)
        zzg = jax.nn.sigmoid(iz + hz)
        nn = jnp.tanh(inn + r * hn)
        gfn_o[...] = (1.0 - zzg) * nn + zzg * gfb


def _tc4(gf, Wg, u, hvr, gid3, gru_w):
    nblk = N // NBLK
    return pl.pallas_call(
        _tc4_body,
        grid=(nblk,),
        in_specs=[
            pl.BlockSpec((G, GD), lambda i: (0, 0)),
            pl.BlockSpec((GD, 1), lambda i: (0, 0)),
            pl.BlockSpec((NBLK, 1), lambda i: (i, 0)),
            pl.BlockSpec((NBLK, GD), lambda i: (i, 0)),
            pl.BlockSpec((1, 1, NBLK), lambda i: (i, 0, 0)),
            *[pl.BlockSpec((GD, GD), lambda i: (0, 0)) for _ in range(6)],
            pl.BlockSpec((1, 3 * GD), lambda i: (0, 0)),
            pl.BlockSpec((1, 3 * GD), lambda i: (0, 0)),
        ],
        out_specs=pl.BlockSpec((G, GD), lambda i: (0, 0)),
        out_shape=jax.ShapeDtypeStruct((G, GD), jnp.float32),
        scratch_shapes=[
            pltpu.VMEM((G, GD), jnp.float32),
            pltpu.VMEM((G, 1), jnp.float32),
        ],
    )(gf, Wg, u, hvr, gid3, *gru_w)


def _tc5_body(gf, pw0, pw1, pw2, pw3, pw4, s0, s1, s2, s3, s4, pb, out_o):
    gfb = gf[...]
    acc = jnp.zeros((S, 1), jnp.float32)
    for pw, sm in ((pw0, s0), (pw1, s1), (pw2, s2), (pw3, s3), (pw4, s4)):
        ui = jnp.dot(gfb, pw[...], preferred_element_type=jnp.float32)
        oh = (sm[...] == lax.broadcasted_iota(jnp.int32, (S, G), 1)).astype(jnp.float32)
        acc = acc + jnp.dot(oh, ui, preferred_element_type=jnp.float32)
    out_o[...] = acc + pb[...]


def _tc5(gf, pws, sms, pb):
    return pl.pallas_call(
        _tc5_body,
        in_specs=[
            pl.BlockSpec((G, GD), lambda: (0, 0)),
            *[pl.BlockSpec((GD, 1), lambda: (0, 0)) for _ in range(5)],
            *[pl.BlockSpec((S, 1), lambda: (0, 0)) for _ in range(5)],
            pl.BlockSpec((1, 1), lambda: (0, 0)),
        ],
        out_specs=pl.BlockSpec((S, 1), lambda: (0, 0)),
        out_shape=jax.ShapeDtypeStruct((S, 1), jnp.float32),
    )(gf, *pws, *sms, pb)


def _pad_rows(w, rows):
    return jnp.pad(w, ((0, rows - w.shape[0]), (0, 0)))


def _pad_cols(w, cols):
    return jnp.pad(w, ((0, 0), (0, cols - w.shape[1])))


def _gru_weights(Wih, Whh, bih, bhh, xdim):
    WirT = _pad_rows(Wih[0:GD].T, xdim)
    WizT = _pad_rows(Wih[GD:2 * GD].T, xdim)
    WinT = _pad_rows(Wih[2 * GD:].T, xdim)
    WhrT = Whh[0:GD].T
    WhzT = Whh[GD:2 * GD].T
    WhnT = Whh[2 * GD:].T
    return (WirT, WizT, WinT, WhrT, WhzT, WhnT,
            bih[None, :], bhh[None, :])


def kernel(node_feats, edge_feats, edge_index, node_graph_ids, samples, gc_pn_W, gc_pn_b, gc_pe1_W, gc_pe1_b, gc_pe2_W, gc_pe2_b, gc_et_W, gc_et_b, gc_gru_Wih, gc_gru_Whh, gc_gru_bih, gc_gru_bhh, gl_pe_W, gl_pe_b, gl_pn_W, gl_pn_b, gl_gru_Wih, gl_gru_Whh, gl_gru_bih, gl_gru_bhh, ro0_cl_W, ro0_cl_b, ro0_pn_W, ro0_pn_b, ro0_gru_Wih, ro0_gru_Whh, ro0_gru_bih, ro0_gru_bhh, ro1_cl_W, ro1_cl_b, ro1_pn_W, ro1_pn_b, ro1_gru_Wih, ro1_gru_Whh, ro1_gru_bih, ro1_gru_bhh, pred_W, pred_b):
    src = jnp.pad(edge_index[0], (0, E_PAD - E))
    dst = jnp.pad(edge_index[1], (0, E_PAD - E), constant_values=N)
    ef_pad = jnp.pad(edge_feats, ((0, E_PAD - E), (0, 0)))
    gid3 = node_graph_ids.reshape(N // NBLK, 1, NBLK)

    pnWT = gc_pn_W.T
    pnb = gc_pn_b[None, :]
    W1aT = _pad_cols(gc_pe1_W[:, :NFS].T, D)
    W1bT = _pad_cols(gc_pe1_W[:, NFS:].T, D)
    b1 = _pad_cols(gc_pe1_b[None, :], D)
    Wa = gc_pe2_W[:, :GD].T
    wb = jnp.pad(gc_pe2_W[0, GD:], (0, D - GD))
    pe2b = gc_pe2_b[None, :]
    etWT = _pad_rows(gc_et_W.T, D)
    etb = gc_et_b[None, :]
    gc_gru = _gru_weights(gc_gru_Wih, gc_gru_Whh, gc_gru_bih, gc_gru_bhh, GD)
    Wd = gl_pe_W[:, :GD].T
    Ws = gl_pe_W[:, GD:].T
    peb = gl_pe_b[None, :]
    pnT = _pad_cols(gl_pn_W.T, D)
    pnT0, pnT1 = pnT[:, :LW], pnT[:, GD - LW + 8:]
    pnb2 = _pad_cols(gl_pn_b[None, :], D)
    pnb0, pnb1 = pnb2[:, :LW], pnb2[:, GD - LW + 8:]
    gl_gru = _gru_weights(gl_gru_Wih, gl_gru_Whh, gl_gru_bih, gl_gru_bhh, D)
    ro_w = (ro0_cl_W[:, GD:].T, ro0_cl_b[None, :], ro0_pn_W.T, ro0_pn_b[None, :],
            ro1_cl_W[:, GD:].T, ro1_cl_b[None, :], ro1_pn_W.T, ro1_pn_b[None, :])
    Wg0 = ro0_cl_W[:, :GD].T
    Wg1 = ro1_cl_W[:, :GD].T
    ro0_gru = _gru_weights(ro0_gru_Wih, ro0_gru_Whh, ro0_gru_bih, ro0_gru_bhh, GD)
    ro1_gru = _gru_weights(ro1_gru_Wih, ro1_gru_Whh, ro1_gru_bih, ro1_gru_bhh, GD)
    pw5 = pred_W.reshape(5, GD)
    pws = [pw5[i][:, None] for i in range(5)]
    sms = [samples[:, i][:, None] for i in range(5)]

    hv_new, hx, q1 = _tc1_node(node_feats, pnWT, pnb, W1aT, Wa, pe2b)
    EX = _tc1_edge(ef_pad, W1bT, b1)

    s1p = _sc1(src, dst, hx, EX, jnp.pad(q1.reshape(N), (0, NA - N)), wb)
    s1full = jnp.concatenate([s1p[0], s1p[1][:, 16:]], axis=1)

    nf, qd, qs, hvp0, hvp1 = _tc2(s1full, hv_new, etWT, etb, gc_gru,
                                  Wd, Ws, peb, pnT0, pnb0, pnT1, pnb1)

    s2p = _sc2(src, dst, hvp0, hvp1, jnp.pad(qd.reshape(N), (0, NA - N)),
               jnp.pad(qs.reshape(N), (0, NA - N)))
    s2full = jnp.concatenate([s2p[0], s2p[1][:, 16:]], axis=1)

    u0, hvr0, u1, hvr1, gf0 = _tc3(s2full, nf, gid3, gl_gru, ro_w)

    gf1 = _tc4(gf0, Wg0, u0, hvr0, gid3, ro0_gru)
    gf2 = _tc4(gf1, Wg1, u1, hvr1, gid3, ro1_gru)

    return _tc5(gf2, pws, sms, pred_b[None, :])
